# baseline (speedup 1.0000x reference)
import dataclasses
import functools

import jax
import jax.numpy as jnp
from jax import lax
from jax.experimental import pallas as pl
from jax.experimental.pallas import tpu as pltpu
from jax.experimental.pallas import tpu_sc as plsc

NCLS = 200
CPAD = 208
NCHUNK = CPAD // 16
TABW = 4 * CPAD
NBATCH = 16384
NTILES = 16
PER_TILE = NBATCH // NTILES
NVEC = PER_TILE // 16


def _sc_center_loss():
    mesh = plsc.VectorSubcoreMesh(core_axis_name="c", subcore_axis_name="s")
    cparams = pltpu.CompilerParams()
    if "needs_layout_passes" in pltpu.CompilerParams.__dataclass_fields__:
        cparams = dataclasses.replace(cparams, needs_layout_passes=False)

    @functools.partial(
        pl.kernel,
        compiler_params=cparams,
        out_type=jax.ShapeDtypeStruct((16,), jnp.float32),
        mesh=mesh,
        scratch_types=[
            pltpu.VMEM((2 * PER_TILE,), jnp.float32),
            pltpu.VMEM((PER_TILE,), jnp.int32),
            pltpu.VMEM((1, TABW), jnp.float32),
            pltpu.VMEM((416,), jnp.float32),
            pltpu.VMEM((1,), jnp.int32),
            pltpu.VMEM((16,), jnp.float32),
            pltpu.VMEM((1, 16), jnp.float32),
            pltpu.VMEM_SHARED((1, TABW), jnp.float32),
            pltpu.VMEM_SHARED((1, 16), jnp.float32),
            pltpu.SemaphoreType.DMA,
            pltpu.SemaphoreType.DMA,
            pltpu.SemaphoreType.DMA,
        ],
    )
    def sc_kernel(feat_hbm, lbl_hbm, lam_hbm, ctr_hbm, idx_hbm, out_hbm,
                  feat_v, lbl_v, tab_v, ctr_v, idx_v, lam_v, res_v,
                  tab_sh, part_sh, sem_a, sem_c, sem_l):
        cid = lax.axis_index("c")
        sid = lax.axis_index("s")

        @pl.when(cid == 0)
        def _():
            lanes = lax.iota(jnp.int32, 16)
            zi = lanes * 0
            zf = jnp.zeros((16,), jnp.float32)

            base = sid * PER_TILE
            h_lbl = pltpu.async_copy(
                lbl_hbm.at[pl.ds(base, PER_TILE)], lbl_v, sem_a)
            h_feat = pltpu.async_copy(
                feat_hbm.at[pl.ds(2 * base, 2 * PER_TILE)], feat_v, sem_a)
            h_idx = pltpu.async_copy(idx_hbm, idx_v, sem_a)

            @pl.when(sid < NCHUNK)
            def _():
                pltpu.async_copy(ctr_hbm, ctr_v.at[pl.ds(0, 2 * NCLS)], sem_c)

            @pl.when(sid == 0)
            def _():
                pltpu.async_copy(lam_hbm, lam_v, sem_l)

            @plsc.parallel_loop(0, TABW // 16, unroll=4)
            def _(j):
                tab_v[0, pl.ds(j * 16, 16)] = zf

            @pl.when(sid == 0)
            def _():
                pltpu.sync_copy(tab_v, tab_sh)
                res_v[0, pl.ds(0, 16)] = zf
                pltpu.sync_copy(res_v, part_sh)

            h_lbl.wait()
            h_feat.wait()
            h_idx.wait()
            plsc.subcore_barrier()

            onesf = zf + 1.0

            @plsc.parallel_loop(0, NVEC, unroll=8)
            def _(i):
                lab = lbl_v[pl.ds(i * 16, 16)]
                rows = lanes * 2 + i * 32
                fx = plsc.load_gather(feat_v, [rows])
                fy = plsc.load_gather(feat_v, [rows + 1])
                ff = fx * fx + fy * fy
                plsc.addupdate_scatter(tab_v, [zi, lab], onesf)
                plsc.addupdate_scatter(tab_v, [zi, lab + CPAD], fx)
                plsc.addupdate_scatter(tab_v, [zi, lab + 2 * CPAD], fy)
                plsc.addupdate_scatter(tab_v, [zi, lab + 3 * CPAD], ff)

            pltpu.sync_copy(tab_v, tab_sh.at[idx_v], add=True)
            plsc.subcore_barrier()

            @pl.when(sid < NCHUNK)
            def _():
                pltpu.make_async_copy(
                    ctr_hbm, ctr_v.at[pl.ds(0, 2 * NCLS)], sem_c).wait()
                ctr_v[pl.ds(2 * NCLS, 16)] = zf
                pltpu.sync_copy(tab_sh, tab_v)
                c0 = sid * 16
                n = tab_v[0, pl.ds(c0, 16)]
                sx = tab_v[0, pl.ds(c0 + CPAD, 16)]
                sy = tab_v[0, pl.ds(c0 + 2 * CPAD, 16)]
                s1 = tab_v[0, pl.ds(c0 + 3 * CPAD, 16)]
                cls2 = (lanes + c0) * 2
                cx = plsc.load_gather(ctr_v, [cls2])
                cy = plsc.load_gather(ctr_v, [cls2 + 1])
                contrib = (s1 - 2.0 * (cx * sx + cy * sy)) / jnp.maximum(n, 1.0)
                contrib = contrib + jnp.where(n > 0.0, cx * cx + cy * cy, 0.0)
                res_v[0, pl.ds(0, 16)] = contrib
                pltpu.sync_copy(res_v, part_sh.at[idx_v], add=True)

            plsc.subcore_barrier()

            @pl.when(sid == 0)
            def _():
                pltpu.make_async_copy(lam_hbm, lam_v, sem_l).wait()
                pltpu.sync_copy(part_sh, res_v)
                acc = res_v[0, pl.ds(0, 16)]
                res_v[0, pl.ds(0, 16)] = (
                    (zf + jnp.sum(acc)) * lam_v[...] * (0.5 / NBATCH))
                pltpu.sync_copy(res_v.at[0], out_hbm)

    return sc_kernel


_SC_KERNEL = _sc_center_loss()


def kernel(feature, label, lambdas, center):
    feat_flat = feature.reshape(2 * NBATCH)
    ctr_flat = center.reshape(2 * NCLS)
    lam = jnp.full((16,), lambdas, dtype=jnp.float32)
    row0 = jnp.zeros((1,), jnp.int32)
    out = _SC_KERNEL(feat_flat, label, lam, ctr_flat, row0)
    return out[0]


# speedup vs baseline: 1.0064x; 1.0064x over previous
import dataclasses
import functools

import jax
import jax.numpy as jnp
from jax import lax
from jax.experimental import pallas as pl
from jax.experimental.pallas import tpu as pltpu
from jax.experimental.pallas import tpu_sc as plsc

NCLS = 200
CPAD = 208
NCHUNK = CPAD // 16
TABW = 4 * CPAD
NBATCH = 16384
NTILES = 16
PER_TILE = NBATCH // NTILES
NVEC = PER_TILE // 16


def _sc_center_loss():
    mesh = plsc.VectorSubcoreMesh(core_axis_name="c", subcore_axis_name="s")
    cparams = pltpu.CompilerParams()
    if "needs_layout_passes" in pltpu.CompilerParams.__dataclass_fields__:
        cparams = dataclasses.replace(cparams, needs_layout_passes=False)

    @functools.partial(
        pl.kernel,
        compiler_params=cparams,
        out_type=jax.ShapeDtypeStruct((16,), jnp.float32),
        mesh=mesh,
        scratch_types=[
            pltpu.VMEM((2 * PER_TILE,), jnp.float32),
            pltpu.VMEM((PER_TILE,), jnp.int32),
            pltpu.VMEM((1, TABW), jnp.float32),
            pltpu.VMEM((416,), jnp.float32),
            pltpu.VMEM((1,), jnp.int32),
            pltpu.VMEM((16,), jnp.float32),
            pltpu.VMEM((1, 16), jnp.float32),
            pltpu.VMEM_SHARED((1, TABW), jnp.float32),
            pltpu.VMEM_SHARED((1, 16), jnp.float32),
            pltpu.SemaphoreType.DMA,
            pltpu.SemaphoreType.DMA,
            pltpu.SemaphoreType.DMA,
        ],
    )
    def sc_kernel(feat_hbm, lbl_hbm, lam_hbm, ctr_hbm, idx_hbm, out_hbm,
                  feat_v, lbl_v, tab_v, ctr_v, idx_v, lam_v, res_v,
                  tab_sh, part_sh, sem_a, sem_c, sem_l):
        cid = lax.axis_index("c")
        sid = lax.axis_index("s")

        @pl.when(cid == 0)
        def _():
            lanes = lax.iota(jnp.int32, 16)
            zi = lanes * 0
            zf = jnp.zeros((16,), jnp.float32)

            base = sid * PER_TILE
            h_lbl = pltpu.async_copy(
                lbl_hbm.at[pl.ds(base, PER_TILE)], lbl_v, sem_a)
            h_feat = pltpu.async_copy(
                feat_hbm.at[pl.ds(2 * base, 2 * PER_TILE)], feat_v, sem_a)
            h_idx = pltpu.async_copy(idx_hbm, idx_v, sem_a)

            @pl.when(sid < NCHUNK)
            def _():
                pltpu.async_copy(ctr_hbm, ctr_v.at[pl.ds(0, 2 * NCLS)], sem_c)

            @pl.when(sid == 0)
            def _():
                pltpu.async_copy(lam_hbm, lam_v, sem_l)

            @plsc.parallel_loop(0, TABW // 16, unroll=4)
            def _(j):
                tab_v[0, pl.ds(j * 16, 16)] = zf

            @pl.when(sid == 0)
            def _():
                pltpu.sync_copy(tab_v, tab_sh)
                res_v[0, pl.ds(0, 16)] = zf
                pltpu.sync_copy(res_v, part_sh)

            h_lbl.wait()
            h_feat.wait()
            h_idx.wait()
            plsc.subcore_barrier()

            onesf = zf + 1.0

            @plsc.parallel_loop(0, NVEC, unroll=4)
            def _(i):
                lab = lbl_v[pl.ds(i * 16, 16)]
                rows = lanes * 2 + i * 32
                fx = plsc.load_gather(feat_v, [rows])
                fy = plsc.load_gather(feat_v, [rows + 1])
                ff = fx * fx + fy * fy
                plsc.addupdate_scatter(tab_v, [zi, lab], onesf)
                plsc.addupdate_scatter(tab_v, [zi, lab + CPAD], fx)
                plsc.addupdate_scatter(tab_v, [zi, lab + 2 * CPAD], fy)
                plsc.addupdate_scatter(tab_v, [zi, lab + 3 * CPAD], ff)

            pltpu.sync_copy(tab_v, tab_sh.at[idx_v], add=True)
            plsc.subcore_barrier()

            @pl.when(sid < NCHUNK)
            def _():
                pltpu.make_async_copy(
                    ctr_hbm, ctr_v.at[pl.ds(0, 2 * NCLS)], sem_c).wait()
                ctr_v[pl.ds(2 * NCLS, 16)] = zf
                pltpu.sync_copy(tab_sh, tab_v)
                c0 = sid * 16
                n = tab_v[0, pl.ds(c0, 16)]
                sx = tab_v[0, pl.ds(c0 + CPAD, 16)]
                sy = tab_v[0, pl.ds(c0 + 2 * CPAD, 16)]
                s1 = tab_v[0, pl.ds(c0 + 3 * CPAD, 16)]
                cls2 = (lanes + c0) * 2
                cx = plsc.load_gather(ctr_v, [cls2])
                cy = plsc.load_gather(ctr_v, [cls2 + 1])
                contrib = (s1 - 2.0 * (cx * sx + cy * sy)) / jnp.maximum(n, 1.0)
                contrib = contrib + jnp.where(n > 0.0, cx * cx + cy * cy, 0.0)
                res_v[0, pl.ds(0, 16)] = contrib
                pltpu.sync_copy(res_v, part_sh.at[idx_v], add=True)

            plsc.subcore_barrier()

            @pl.when(sid == 0)
            def _():
                pltpu.make_async_copy(lam_hbm, lam_v, sem_l).wait()
                pltpu.sync_copy(part_sh, res_v)
                acc = res_v[0, pl.ds(0, 16)]
                res_v[0, pl.ds(0, 16)] = (
                    (zf + jnp.sum(acc)) * lam_v[...] * (0.5 / NBATCH))
                pltpu.sync_copy(res_v.at[0], out_hbm)

    return sc_kernel


_SC_KERNEL = _sc_center_loss()


def kernel(feature, label, lambdas, center):
    feat_flat = feature.reshape(2 * NBATCH)
    ctr_flat = center.reshape(2 * NCLS)
    lam = jnp.full((16,), lambdas, dtype=jnp.float32)
    row0 = jnp.zeros((1,), jnp.int32)
    out = _SC_KERNEL(feat_flat, label, lam, ctr_flat, row0)
    return out[0]
